# Initial kernel scaffold; baseline (speedup 1.0000x reference)
import jax
import jax.numpy as jnp
from jax.experimental import pallas as pl


def kernel(unary, binary, index1, index2, unary_clause_weights, binary_clause_weights):
    raise NotImplementedError("write your pallas kernel here")


# speedup vs baseline: 17.3910x; 17.3910x over previous
import functools

import jax
import jax.numpy as jnp
from jax import lax
from jax.experimental import pallas as pl
from jax.experimental.pallas import tpu as pltpu
from jax.experimental.pallas import tpu_sc as plsc

N_NODES = 100000
N_EDGES = 1600000
N_UNARY = 6

ROW_W = 8
TC_BLK = 2048
NPAD = 102400
JUNK = N_NODES

NC, NS = 2, 16
NW = NC * NS
GRP = 128
KSUB = 4
CHUNK = KSUB * GRP
CHUNKS_PER_W = 98
EPAD = NW * CHUNKS_PER_W * CHUNK
ROWS_PER_SUB = NPAD // NS


def _unary_kernel(w_ref, x_ref, o_ref):
  x = x_ref[...]
  o_ref[...] = x
  for c, wi in ((0, w_ref[0]), (2, w_ref[1])):
    a = x[:, c:c + 1]
    b = x[:, c + 1:c + 2]
    ea = jnp.exp(-a)
    eb = jnp.exp(b)
    s = ea + eb
    o_ref[:, c:c + 1] = a - wi * ea / s
    o_ref[:, c + 1:c + 2] = b + wi * eb / s


def _combine_kernel(u_ref, p0_ref, p1_ref, o_ref):
  o_ref[...] = u_ref[...] + p0_ref[...] + p1_ref[...]


def _edge_kernel(u8, idxc, bpad, wpad, zeros8, p_out, bout,
                 idx_v0, idx_v1, b_v0, b_v1, rows1_0, rows1_1, rows2_0,
                 rows2_1, stage1, stage2, bout_v, w_v, acc_s,
                 sem_i0, sem_i1, sem_r0, sem_r1):
  cid = lax.axis_index("c")
  sid = lax.axis_index("s")
  wid = cid * NS + sid
  c_base = wid * CHUNKS_PER_W

  idx_v = (idx_v0, idx_v1)
  b_v = (b_v0, b_v1)
  rows1 = (rows1_0, rows1_1)
  rows2 = (rows2_0, rows2_1)
  sem_i = (sem_i0, sem_i1)
  sem_r = (sem_r0, sem_r1)

  sub0 = sid * ROWS_PER_SUB
  pltpu.sync_copy(zeros8.at[pl.ds(sub0, ROWS_PER_SUB)],
                  acc_s.at[pl.ds(sub0, ROWS_PER_SUB)])
  pltpu.sync_copy(zeros8.at[pl.ds(0, CHUNK)], stage1)
  pltpu.sync_copy(zeros8.at[pl.ds(0, CHUNK)], stage2)
  pltpu.sync_copy(wpad, w_v)
  plsc.subcore_barrier()

  lanes = lax.iota(jnp.int32, 16)
  wvecs = [w_v[i, :] for i in range(6)]

  def issue_idx(c, s):
    r0 = (c_base + c) * KSUB
    pltpu.async_copy(idxc.at[pl.ds(r0, KSUB)], idx_v[s], sem_i[s])
    pltpu.async_copy(bpad.at[pl.ds(r0 * GRP, CHUNK)], b_v[s], sem_i[s])

  def wait_idx(s):
    pltpu.make_async_copy(idxc.at[pl.ds(0, KSUB)], idx_v[s], sem_i[s]).wait()
    pltpu.make_async_copy(bpad.at[pl.ds(0, CHUNK)], b_v[s], sem_i[s]).wait()

  def issue_gathers(s):
    for k in range(KSUB):
      pltpu.async_copy(u8.at[idx_v[s].at[k, 0]],
                       rows1[s].at[pl.ds(k * GRP, GRP)], sem_r[s])
      pltpu.async_copy(u8.at[idx_v[s].at[k, 1]],
                       rows2[s].at[pl.ds(k * GRP, GRP)], sem_r[s])

  def wait_gathers(s):
    for k in range(KSUB):
      pltpu.make_async_copy(u8.at[idx_v[s].at[k, 0]],
                            rows1[s].at[pl.ds(k * GRP, GRP)], sem_r[s]).wait()
      pltpu.make_async_copy(u8.at[idx_v[s].at[k, 1]],
                            rows2[s].at[pl.ds(k * GRP, GRP)], sem_r[s]).wait()

  def compute(s):
    def grp_body(g, carry2):
      row = g * 16 + lanes
      b16 = b_v[s][pl.ds(g * 16, 16)]
      eb = jnp.exp(-b16)
      db = b16
      for i in range(6):
        coli = jnp.full((16,), i, jnp.int32)
        u1 = plsc.load_gather(rows1[s], [row, coli])
        u2 = plsc.load_gather(rows2[s], [row, coli])
        e1 = jnp.exp(-u1)
        e2 = jnp.exp(u2)
        inv = wvecs[i] / (e1 + eb + e2)
        plsc.store_scatter(stage1, [row, coli], -e1 * inv)
        plsc.store_scatter(stage2, [row, coli], e2 * inv)
        db = db - eb * inv
      bout_v[pl.ds(g * 16, 16)] = db
      return carry2
    lax.fori_loop(0, CHUNK // 16, grp_body, 0)

  def scatter_add(s):
    for k in range(KSUB):
      pltpu.sync_copy(stage1.at[pl.ds(k * GRP, GRP)],
                      acc_s.at[idx_v[s].at[k, 0]], add=True)
      pltpu.sync_copy(stage2.at[pl.ds(k * GRP, GRP)],
                      acc_s.at[idx_v[s].at[k, 1]], add=True)

  def store_bout(c):
    e0 = (c_base + c) * CHUNK
    pltpu.sync_copy(bout_v, bout.at[pl.ds(e0, CHUNK)])

  issue_idx(0, 0)
  issue_idx(1, 1)
  wait_idx(0)
  issue_gathers(0)

  def pair_body(p, carry):
    for (off, s, o) in ((0, 0, 1), (1, 1, 0)):
      c = 2 * p + off

      @pl.when(c + 1 < CHUNKS_PER_W)
      def _():
        wait_idx(o)
        issue_gathers(o)

      wait_gathers(s)
      compute(s)
      scatter_add(s)
      store_bout(c)

      @pl.when(c + 2 < CHUNKS_PER_W)
      def _():
        issue_idx(c + 2, s)
    return carry

  lax.fori_loop(0, CHUNKS_PER_W // 2, pair_body, 0)

  plsc.subcore_barrier()
  pltpu.sync_copy(acc_s.at[pl.ds(sub0, ROWS_PER_SUB)],
                  p_out.at[cid, pl.ds(sub0, ROWS_PER_SUB)])


_edge_call = functools.partial(
    pl.kernel,
    out_type=[
        jax.ShapeDtypeStruct((NC, NPAD, ROW_W), jnp.float32),
        jax.ShapeDtypeStruct((EPAD,), jnp.float32),
    ],
    mesh=plsc.VectorSubcoreMesh(core_axis_name="c", subcore_axis_name="s"),
    compiler_params=pltpu.CompilerParams(
        needs_layout_passes=False, use_tc_tiling_on_sc=False),
    scratch_types=[
        pltpu.VMEM((KSUB, 2, GRP), jnp.int32),
        pltpu.VMEM((KSUB, 2, GRP), jnp.int32),
        pltpu.VMEM((CHUNK,), jnp.float32),
        pltpu.VMEM((CHUNK,), jnp.float32),
        pltpu.VMEM((CHUNK, ROW_W), jnp.float32),
        pltpu.VMEM((CHUNK, ROW_W), jnp.float32),
        pltpu.VMEM((CHUNK, ROW_W), jnp.float32),
        pltpu.VMEM((CHUNK, ROW_W), jnp.float32),
        pltpu.VMEM((CHUNK, ROW_W), jnp.float32),
        pltpu.VMEM((CHUNK, ROW_W), jnp.float32),
        pltpu.VMEM((CHUNK,), jnp.float32),
        pltpu.VMEM((6, 16), jnp.float32),
        pltpu.VMEM_SHARED((NPAD, ROW_W), jnp.float32),
        pltpu.SemaphoreType.DMA,
        pltpu.SemaphoreType.DMA,
        pltpu.SemaphoreType.DMA,
        pltpu.SemaphoreType.DMA,
    ],
)(_edge_kernel)


@jax.jit
def kernel(unary, binary, index1, index2, unary_clause_weights,
           binary_clause_weights):
  f32 = jnp.float32
  un8 = jnp.pad(unary, ((0, NPAD - N_NODES), (0, ROW_W - N_UNARY)))

  u8 = pl.pallas_call(
      _unary_kernel,
      grid=(NPAD // TC_BLK,),
      in_specs=[
          pl.BlockSpec(memory_space=pltpu.MemorySpace.SMEM),
          pl.BlockSpec((TC_BLK, ROW_W), lambda i: (i, 0)),
      ],
      out_specs=pl.BlockSpec((TC_BLK, ROW_W), lambda i: (i, 0)),
      out_shape=jax.ShapeDtypeStruct((NPAD, ROW_W), f32),
  )(unary_clause_weights, un8)

  idx1p = jnp.pad(index1, (0, EPAD - N_EDGES),
                  constant_values=JUNK).reshape(EPAD // GRP, GRP)
  idx2p = jnp.pad(index2, (0, EPAD - N_EDGES),
                  constant_values=JUNK).reshape(EPAD // GRP, GRP)
  idxc = jnp.stack([idx1p, idx2p], axis=1)
  bpad = jnp.pad(binary[:, 0], (0, EPAD - N_EDGES))
  wpad = jnp.tile(binary_clause_weights[:, None], (1, 16))
  zeros8 = jnp.zeros((NPAD, ROW_W), f32)

  p, dbout = _edge_call(u8, idxc, bpad, wpad, zeros8)

  u_out8 = pl.pallas_call(
      _combine_kernel,
      grid=(NPAD // TC_BLK,),
      in_specs=[pl.BlockSpec((TC_BLK, ROW_W), lambda i: (i, 0))] * 3,
      out_specs=pl.BlockSpec((TC_BLK, ROW_W), lambda i: (i, 0)),
      out_shape=jax.ShapeDtypeStruct((NPAD, ROW_W), f32),
  )(u8, p[0], p[1])

  return (u_out8[:N_NODES, :N_UNARY], dbout[:N_EDGES].reshape(N_EDGES, 1))


# speedup vs baseline: 20.3129x; 1.1680x over previous
import functools

import jax
import jax.numpy as jnp
from jax import lax
from jax.experimental import pallas as pl
from jax.experimental.pallas import tpu as pltpu
from jax.experimental.pallas import tpu_sc as plsc

N_NODES = 100000
N_EDGES = 1600000
N_UNARY = 6

ROW_W = 8
TC_BLK = 2000

NC, NS = 2, 16
NW = NC * NS
GRP = 128
KSUB = 4
CHUNK = KSUB * GRP
N_GROUPS = N_EDGES // GRP
N_CHUNKS = N_GROUPS // KSUB
BASE_CH = N_CHUNKS // NW
EXTRA = N_CHUNKS - BASE_CH * NW
MAX_CH = BASE_CH + 1
ROWS_PER_SUB = N_NODES // NS


def _unary_kernel(w_ref, x_ref, o_ref):
  x = x_ref[...]
  o_ref[:, 4:6] = x[:, 4:6]
  o_ref[:, 6:8] = jnp.zeros_like(x[:, 0:2])
  for c, wi in ((0, w_ref[0]), (2, w_ref[1])):
    a = x[:, c:c + 1]
    b = x[:, c + 1:c + 2]
    ea = jnp.exp(-a)
    eb = jnp.exp(b)
    s = ea + eb
    o_ref[:, c:c + 1] = a - wi * ea / s
    o_ref[:, c + 1:c + 2] = b + wi * eb / s


def _combine_kernel(u_ref, p0_ref, p1_ref, o_ref):
  o_ref[...] = (u_ref[...] + p0_ref[...] + p1_ref[...])[:, :N_UNARY]


def _edge_kernel(u8, idx1r, idx2r, b1d, wpad, zeros8, p_out, bout,
                 i1_q0, i1_q1, i1_q2, i1_q3, i2_q0, i2_q1, i2_q2, i2_q3,
                 b_v0, b_v1, rows1_0, rows1_1, rows2_0, rows2_1,
                 stage1_0, stage1_1, stage2_0, stage2_1, bout_v0, bout_v1,
                 w_v, acc_s,
                 sem_i0, sem_i1, sem_i2, sem_i3, sem_r0, sem_r1,
                 sem_s0, sem_s1, sem_o0, sem_o1):
  cid = lax.axis_index("c")
  sid = lax.axis_index("s")
  wid = cid * NS + sid
  ch0 = BASE_CH * wid + jnp.minimum(wid, EXTRA)
  my_n = jnp.where(wid < EXTRA, BASE_CH + 1, BASE_CH)

  idx1_v = (i1_q0, i1_q1, i1_q2, i1_q3)
  idx2_v = (i2_q0, i2_q1, i2_q2, i2_q3)
  sem_i = (sem_i0, sem_i1, sem_i2, sem_i3)
  b_v = (b_v0, b_v1)
  rows1 = (rows1_0, rows1_1)
  rows2 = (rows2_0, rows2_1)
  stage1 = (stage1_0, stage1_1)
  stage2 = (stage2_0, stage2_1)
  bout_v = (bout_v0, bout_v1)
  sem_r = (sem_r0, sem_r1)
  sem_s = (sem_s0, sem_s1)
  sem_o = (sem_o0, sem_o1)

  sub0 = sid * ROWS_PER_SUB
  pltpu.sync_copy(zeros8.at[pl.ds(sub0, ROWS_PER_SUB)],
                  acc_s.at[pl.ds(sub0, ROWS_PER_SUB)])
  for s in (0, 1):
    pltpu.sync_copy(zeros8.at[pl.ds(0, CHUNK)], stage1[s])
    pltpu.sync_copy(zeros8.at[pl.ds(0, CHUNK)], stage2[s])
  pltpu.sync_copy(wpad, w_v)
  plsc.subcore_barrier()

  lanes = lax.iota(jnp.int32, 16)
  wvecs = [w_v[i, :] for i in range(6)]

  def issue_idx(c, q, s):
    e0 = (ch0 + c) * CHUNK
    for k in range(KSUB):
      pltpu.async_copy(idx1r.at[pl.ds(e0 + k * GRP, GRP)],
                       idx1_v[q].at[k], sem_i[q])
      pltpu.async_copy(idx2r.at[pl.ds(e0 + k * GRP, GRP)],
                       idx2_v[q].at[k], sem_i[q])
    pltpu.async_copy(b1d.at[pl.ds(e0, CHUNK)], b_v[s], sem_i[q])

  def wait_idx(q, s):
    for k in range(KSUB):
      pltpu.make_async_copy(idx1r.at[pl.ds(0, GRP)],
                            idx1_v[q].at[k], sem_i[q]).wait()
      pltpu.make_async_copy(idx2r.at[pl.ds(0, GRP)],
                            idx2_v[q].at[k], sem_i[q]).wait()
    pltpu.make_async_copy(b1d.at[pl.ds(0, CHUNK)], b_v[s], sem_i[q]).wait()

  def issue_gathers(q, s):
    for k in range(KSUB):
      pltpu.async_copy(u8.at[idx1_v[q].at[k]],
                       rows1[s].at[pl.ds(k * GRP, GRP)], sem_r[s])
      pltpu.async_copy(u8.at[idx2_v[q].at[k]],
                       rows2[s].at[pl.ds(k * GRP, GRP)], sem_r[s])

  def wait_gathers(q, s):
    for k in range(KSUB):
      pltpu.make_async_copy(u8.at[idx1_v[q].at[k]],
                            rows1[s].at[pl.ds(k * GRP, GRP)], sem_r[s]).wait()
      pltpu.make_async_copy(u8.at[idx2_v[q].at[k]],
                            rows2[s].at[pl.ds(k * GRP, GRP)], sem_r[s]).wait()

  def compute(s):
    def grp_body(g, carry2):
      row = g * 16 + lanes
      b16 = b_v[s][pl.ds(g * 16, 16)]
      eb = jnp.exp(-b16)
      db = b16
      for i in range(6):
        coli = jnp.full((16,), i, jnp.int32)
        u1 = plsc.load_gather(rows1[s], [row, coli])
        u2 = plsc.load_gather(rows2[s], [row, coli])
        e1 = jnp.exp(-u1)
        e2 = jnp.exp(u2)
        inv = wvecs[i] / (e1 + eb + e2)
        plsc.store_scatter(stage1[s], [row, coli], -e1 * inv)
        plsc.store_scatter(stage2[s], [row, coli], e2 * inv)
        db = db - eb * inv
      bout_v[s][pl.ds(g * 16, 16)] = db
      return carry2
    lax.fori_loop(0, CHUNK // 16, grp_body, 0)

  def issue_scatter(q, s):
    for k in range(KSUB):
      pltpu.async_copy(stage1[s].at[pl.ds(k * GRP, GRP)],
                       acc_s.at[idx1_v[q].at[k]], sem_s[s], add=True)
      pltpu.async_copy(stage2[s].at[pl.ds(k * GRP, GRP)],
                       acc_s.at[idx2_v[q].at[k]], sem_s[s], add=True)

  def wait_scatter(q, s):
    for k in range(KSUB):
      pltpu.make_async_copy(stage1[s].at[pl.ds(k * GRP, GRP)],
                            acc_s.at[idx1_v[q].at[k]], sem_s[s]).wait()
      pltpu.make_async_copy(stage2[s].at[pl.ds(k * GRP, GRP)],
                            acc_s.at[idx2_v[q].at[k]], sem_s[s]).wait()

  def issue_bout(c, s):
    e0 = (ch0 + c) * CHUNK
    pltpu.async_copy(bout_v[s], bout.at[pl.ds(e0, CHUNK)], sem_o[s])

  def wait_bout(s):
    pltpu.make_async_copy(bout_v[s], bout.at[pl.ds(0, CHUNK)], sem_o[s]).wait()

  issue_idx(0, 0, 0)
  issue_idx(1, 1, 1)
  wait_idx(0, 0)
  issue_gathers(0, 0)

  def quad_body(qd, carry):
    for off in range(4):
      q = off
      s = off % 2
      qn = (off + 1) % 4
      sn = (off + 1) % 2
      qp = (off + 2) % 4
      c = 4 * qd + off
      active = c < my_n

      @pl.when(c + 1 < my_n)
      def _():
        wait_idx(qn, sn)
        issue_gathers(qn, sn)

      @pl.when(active)
      def _():
        wait_gathers(q, s)

      @pl.when(jnp.logical_and(active, c >= 2))
      def _():
        wait_scatter(qp, s)
        wait_bout(s)

      @pl.when(active)
      def _():
        compute(s)
        issue_scatter(q, s)
        issue_bout(c, s)

      @pl.when(c + 2 < my_n)
      def _():
        issue_idx(c + 2, qp, s)
    return carry

  lax.fori_loop(0, (MAX_CH + 3) // 4, quad_body, 0)

  for s in (0, 1):
    wait_scatter(s, s)
    wait_bout(s)

  plsc.subcore_barrier()
  pltpu.sync_copy(acc_s.at[pl.ds(sub0, ROWS_PER_SUB)],
                  p_out.at[cid, pl.ds(sub0, ROWS_PER_SUB)])


_edge_call = functools.partial(
    pl.kernel,
    out_type=[
        jax.ShapeDtypeStruct((NC, N_NODES, ROW_W), jnp.float32),
        jax.ShapeDtypeStruct((N_EDGES,), jnp.float32),
    ],
    mesh=plsc.VectorSubcoreMesh(core_axis_name="c", subcore_axis_name="s"),
    compiler_params=pltpu.CompilerParams(
        needs_layout_passes=False, use_tc_tiling_on_sc=False),
    scratch_types=(
        [pltpu.VMEM((KSUB, GRP), jnp.int32)] * 8 +
        [pltpu.VMEM((CHUNK,), jnp.float32)] * 2 +
        [pltpu.VMEM((CHUNK, ROW_W), jnp.float32)] * 4 +
        [pltpu.VMEM((CHUNK, ROW_W), jnp.float32)] * 4 +
        [pltpu.VMEM((CHUNK,), jnp.float32)] * 2 +
        [pltpu.VMEM((6, 16), jnp.float32)] +
        [pltpu.VMEM_SHARED((N_NODES, ROW_W), jnp.float32)] +
        [pltpu.SemaphoreType.DMA] * 10
    ),
)(_edge_kernel)


@jax.jit
def kernel(unary, binary, index1, index2, unary_clause_weights,
           binary_clause_weights):
  f32 = jnp.float32

  u8 = pl.pallas_call(
      _unary_kernel,
      grid=(N_NODES // TC_BLK,),
      in_specs=[
          pl.BlockSpec(memory_space=pltpu.MemorySpace.SMEM),
          pl.BlockSpec((TC_BLK, N_UNARY), lambda i: (i, 0)),
      ],
      out_specs=pl.BlockSpec((TC_BLK, ROW_W), lambda i: (i, 0)),
      out_shape=jax.ShapeDtypeStruct((N_NODES, ROW_W), f32),
  )(unary_clause_weights, unary)

  idx1r = index1
  idx2r = index2
  b1d = binary.reshape(N_EDGES)
  wpad = jnp.tile(binary_clause_weights[:, None], (1, 16))
  zeros8 = jnp.zeros((N_NODES, ROW_W), f32)

  p, dbout = _edge_call(u8, idx1r, idx2r, b1d, wpad, zeros8)

  u_out = pl.pallas_call(
      _combine_kernel,
      grid=(N_NODES // TC_BLK,),
      in_specs=[pl.BlockSpec((TC_BLK, ROW_W), lambda i: (i, 0))] * 3,
      out_specs=pl.BlockSpec((TC_BLK, N_UNARY), lambda i: (i, 0)),
      out_shape=jax.ShapeDtypeStruct((N_NODES, N_UNARY), f32),
  )(u8, p[0], p[1])

  return (u_out, dbout.reshape(N_EDGES, 1))
